# baseline (speedup 1.0000x reference)
import functools

import jax
import jax.numpy as jnp
from jax.experimental import pallas as pl

_GROUP = 4


def _cch_kernel(q_ref, k_ref, m_ref, pdw_ref, out_ref, *, inv_bp, inv_bpd):
    g = pl.program_id(0)
    ones = jnp.ones((3, 1), jnp.float32)
    part = jnp.zeros((), jnp.float32)
    for j in range(_GROUP):
        q = q_ref[j]
        k = k_ref[j]
        qq = jnp.dot(q * q, ones, preferred_element_type=jnp.float32)
        kk = jnp.dot(k * k, ones, preferred_element_type=jnp.float32)
        xy = jnp.dot(q * -2.0, k.T, preferred_element_type=jnp.float32)
        row_min = jnp.min(xy + kk.T, axis=1)
        col_min = jnp.min(xy + qq, axis=0)
        m = m_ref[0, j, 0].reshape(1, 1024)
        pdw = pdw_ref[j]
        cham_x_sum = jnp.sum(row_min) + jnp.sum(qq)
        cham_y_masked = jnp.sum((col_min + kk[:, 0]).reshape(1, 1024) * m)
        part += (cham_x_sum + cham_y_masked) * inv_bp
        part += jnp.sum(pdw * pdw) * inv_bpd

    @pl.when(g == 0)
    def _():
        out_ref[...] = jnp.zeros_like(out_ref)

    out_ref[...] += part[None, None]


def kernel(v, v_pred, mask, pred_dw):
    B, P, D = v.shape
    mb, mn, mc, mh, mw = mask.shape
    kern = functools.partial(
        _cch_kernel, inv_bp=1.0 / (B * P), inv_bpd=1.0 / (B * P * D)
    )
    out = pl.pallas_call(
        kern,
        grid=(B // _GROUP,),
        in_specs=[
            pl.BlockSpec((_GROUP, P, D), lambda g: (g, 0, 0)),
            pl.BlockSpec((_GROUP, P, D), lambda g: (g, 0, 0)),
            pl.BlockSpec(
                (1, mn, mc, mh, mw), lambda g: (g, 0, 0, 0, 0)
            ),
            pl.BlockSpec((_GROUP, P, D), lambda g: (g, 0, 0)),
        ],
        out_specs=pl.BlockSpec((1, 1), lambda g: (0, 0)),
        out_shape=jax.ShapeDtypeStruct((1, 1), jnp.float32),
    )(v_pred, v, mask, pred_dw)
    return out[0, 0]


# speedup vs baseline: 1.0581x; 1.0581x over previous
import functools

import jax
import jax.numpy as jnp
from jax.experimental import pallas as pl


def _cch_kernel(q_ref, k_ref, m_ref, pdw_ref, out_ref, *, inv_bp, inv_bpd):
    b = pl.program_id(0)
    q = q_ref[0]
    k = k_ref[0]
    ones = jnp.ones((3, 1), jnp.float32)
    qq = jnp.dot(q * q, ones, preferred_element_type=jnp.float32)
    kk = jnp.dot(k * k, ones, preferred_element_type=jnp.float32)
    xy = jnp.dot(q * -2.0, k.T, preferred_element_type=jnp.float32)
    row_min = jnp.min(xy + kk.T, axis=1)
    col_min = jnp.min(xy + qq, axis=0)
    m = m_ref[0, 0, 0].reshape(1, 1024)
    pdw = pdw_ref[0]
    cham_x_sum = jnp.sum(row_min) + jnp.sum(qq)
    cham_y_masked = jnp.sum((col_min + kk[:, 0]).reshape(1, 1024) * m)
    part = (cham_x_sum + cham_y_masked) * inv_bp + jnp.sum(pdw * pdw) * inv_bpd

    @pl.when(b == 0)
    def _():
        out_ref[...] = jnp.zeros_like(out_ref)

    out_ref[...] += part[None, None]


def kernel(v, v_pred, mask, pred_dw):
    B, P, D = v.shape
    mb, mn, mc, mh, mw = mask.shape
    kern = functools.partial(
        _cch_kernel, inv_bp=1.0 / (B * P), inv_bpd=1.0 / (B * P * D)
    )
    out = pl.pallas_call(
        kern,
        grid=(B,),
        in_specs=[
            pl.BlockSpec((1, P, D), lambda b: (b, 0, 0)),
            pl.BlockSpec((1, P, D), lambda b: (b, 0, 0)),
            pl.BlockSpec(
                (1, 1, mc, mh, mw), lambda b: (b // mn, b % mn, 0, 0, 0)
            ),
            pl.BlockSpec((1, P, D), lambda b: (b, 0, 0)),
        ],
        out_specs=pl.BlockSpec((1, 1), lambda b: (0, 0)),
        out_shape=jax.ShapeDtypeStruct((1, 1), jnp.float32),
    )(v_pred, v, mask, pred_dw)
    return out[0, 0]


# speedup vs baseline: 1.1016x; 1.0411x over previous
import functools

import jax
import jax.numpy as jnp
from jax.experimental import pallas as pl


def _cch_kernel(q_ref, k_ref, m_ref, pdw_ref, out_ref, *, inv_bp, inv_bpd):
    b = pl.program_id(0)
    q = q_ref[0]
    k = k_ref[0]
    ones31 = jnp.ones((3, 1), jnp.float32)
    qq = jnp.dot(q * q, ones31, preferred_element_type=jnp.float32)
    k3 = k.T
    kk_row = jnp.sum(k3 * k3, axis=0, keepdims=True)
    a_aug = jnp.concatenate(
        [q * -2.0, qq, jnp.ones((1024, 1), jnp.float32)], axis=1
    )
    b_aug = jnp.concatenate(
        [k3, jnp.ones((1, 1024), jnp.float32), kk_row], axis=0
    )
    d = jnp.dot(a_aug, b_aug, preferred_element_type=jnp.float32)
    cham_x = jnp.min(d, axis=1)
    cham_y = jnp.min(d, axis=0)
    m = m_ref[0, 0, 0].reshape(1, 1024)
    pdw = pdw_ref[0]
    cham_y_masked = jnp.sum(cham_y.reshape(1, 1024) * m)
    part = (jnp.sum(cham_x) + cham_y_masked) * inv_bp
    part = part + jnp.sum(pdw * pdw) * inv_bpd

    @pl.when(b == 0)
    def _():
        out_ref[...] = jnp.zeros_like(out_ref)

    out_ref[...] += part[None, None]


def kernel(v, v_pred, mask, pred_dw):
    B, P, D = v.shape
    mb, mn, mc, mh, mw = mask.shape
    kern = functools.partial(
        _cch_kernel, inv_bp=1.0 / (B * P), inv_bpd=1.0 / (B * P * D)
    )
    out = pl.pallas_call(
        kern,
        grid=(B,),
        in_specs=[
            pl.BlockSpec((1, P, D), lambda b: (b, 0, 0)),
            pl.BlockSpec((1, P, D), lambda b: (b, 0, 0)),
            pl.BlockSpec(
                (1, 1, mc, mh, mw), lambda b: (b // mn, b % mn, 0, 0, 0)
            ),
            pl.BlockSpec((1, P, D), lambda b: (b, 0, 0)),
        ],
        out_specs=pl.BlockSpec((1, 1), lambda b: (0, 0)),
        out_shape=jax.ShapeDtypeStruct((1, 1), jnp.float32),
    )(v_pred, v, mask, pred_dw)
    return out[0, 0]
